# Initial kernel scaffold; baseline (speedup 1.0000x reference)
import jax
import jax.numpy as jnp
from jax.experimental import pallas as pl


def kernel(in_tensor, xyt_table, yzt_table, xzt_table, W, b):
    raise NotImplementedError("write your pallas kernel here")


# speedup vs baseline: 1.1645x; 1.1645x over previous
import functools

import jax
import jax.numpy as jnp
import numpy as np
from jax import lax
from jax.experimental import pallas as pl
from jax.experimental.pallas import tpu as pltpu
from jax.experimental.pallas import tpu_sc as plsc

N = 524288
L = 16
F = 2
HASH_MASK = np.uint32(2**19 - 1)
P1 = np.uint32(2654435761)
P2 = np.uint32(805459861)
RES = [8, 11, 16, 24, 35, 50, 73, 106, 153, 222, 322, 466, 675, 977, 1415, 2048]
OFF = [0, 729, 2457, 7370, 22995, 69651, 202302, 607526, 1131814, 1656102,
       2180390, 2704678, 3228966, 3753254, 4277542, 4801830]
TOTAL = 5326118
TOTAL_PAD = 5326120
N_DENSE_LOCAL = 4
DENSE_ROWS = OFF[N_DENSE_LOCAL]
DENSE_ELEMS = 2 * DENSE_ROWS + 2

NC = 2
NS = 16
NW = NC * NS
PPW = N // NW
C = 256
G = C // 16
CHUNKS = PPW // C
SLAB = 128
NSLAB = (2 * 8 * C) // SLAB


def _corner_prep(xv, yv, zv, r):
    rf = float(r)
    out = []
    for v in (xv, yv, zv):
        v01 = jnp.clip((v + 1.0) * 0.5, 0.0, 1.0)
        pv = v01 * rf
        iv = pv.astype(jnp.int32)
        fv = pv - iv.astype(jnp.float32)
        c0 = iv
        c1 = jnp.minimum(iv + 1, r)
        out.append((c0, c1, 1.0 - fv, fv))
    return out


def _sc_encode(coords, tables):
    mesh = plsc.VectorSubcoreMesh(core_axis_name="c", subcore_axis_name="s",
                                  num_cores=NC, num_subcores=NS)

    @functools.partial(
        pl.kernel,
        out_type=jax.ShapeDtypeStruct((N, 3 * L * F), jnp.float32),
        mesh=mesh,
        scratch_types=[
            pltpu.VMEM((3, C), jnp.float32),
            pltpu.VMEM((DENSE_ELEMS,), jnp.float32),
            pltpu.VMEM((NSLAB, SLAB), jnp.int32),
            pltpu.VMEM((8 * C,), jnp.float32),
            pltpu.VMEM((2 * 8 * C,), jnp.float32),
            pltpu.VMEM((C, 2 * L), jnp.float32),
            pltpu.SemaphoreType.DMA,
        ],
        compiler_params=pltpu.CompilerParams(
            use_tc_tiling_on_sc=False, needs_layout_passes=False
        ),
    )
    def enc(coords_hbm, tables_hbm, out_hbm, cb, dense, idxb, wb, rows, tstage, sem):
        wid = lax.axis_index("s") * NC + lax.axis_index("c")
        iota16 = lax.iota(jnp.int32, 16)

        def level_dense_local(l):
            r = RES[l]
            rp1 = r + 1
            off2 = 2 * OFF[l]

            def body(g, carry):
                s = g * 16
                xv = cb[0, pl.ds(s, 16)]
                yv = cb[1, pl.ds(s, 16)]
                zv = cb[2, pl.ds(s, 16)]
                (cx0, cx1, wx0, wx1), (cy0, cy1, wy0, wy1), (cz0, cz1, wz0, wz1) = \
                    _corner_prep(xv, yv, zv, r)
                az0 = cz0 * rp1
                az1 = cz1 * rp1
                w00 = wx0 * wy0
                w01 = wx0 * wy1
                w10 = wx1 * wy0
                w11 = wx1 * wy1
                acc0 = jnp.zeros((16,), jnp.float32)
                acc1 = jnp.zeros((16,), jnp.float32)
                for (cx, wxy2) in ((cx0, (w00, w01)), (cx1, (w10, w11))):
                    for (cy, wxy) in ((cy0, wxy2[0]), (cy1, wxy2[1])):
                        for (az, wz) in ((az0, wz0), (az1, wz1)):
                            e0 = ((cy + az) * rp1 + cx) * 2 + off2
                            f0 = plsc.load_gather(dense, [e0])
                            f1 = plsc.load_gather(dense, [e0 + 1])
                            w = wxy * wz
                            acc0 = acc0 + w * f0
                            acc1 = acc1 + w * f1
                riv = iota16 + s
                plsc.store_scatter(tstage, [riv, iota16 * 0 + (2 * l)], acc0)
                plsc.store_scatter(tstage, [riv, iota16 * 0 + (2 * l + 1)], acc1)
                return carry

            lax.fori_loop(0, G, body, 0)

        def level_streamed(l, elem_base):
            r = RES[l]
            rp1 = r + 1
            hashed = (rp1 ** 3) > 2**19

            def pass_a(g, carry):
                s = g * 16
                xv = cb[0, pl.ds(s, 16)]
                yv = cb[1, pl.ds(s, 16)]
                zv = cb[2, pl.ds(s, 16)]
                (cx0, cx1, wx0, wx1), (cy0, cy1, wy0, wy1), (cz0, cz1, wz0, wz1) = \
                    _corner_prep(xv, yv, zv, r)
                w00 = wx0 * wy0
                w01 = wx0 * wy1
                w10 = wx1 * wy0
                w11 = wx1 * wy1
                off = elem_base + 2 * OFF[l]
                if hashed:
                    xs = (cx0, cx1)
                    ys = ((cy0.astype(jnp.uint32) * P1).astype(jnp.int32),
                          (cy1.astype(jnp.uint32) * P1).astype(jnp.int32))
                    zs = ((cz0.astype(jnp.uint32) * P2).astype(jnp.int32),
                          (cz1.astype(jnp.uint32) * P2).astype(jnp.int32))
                else:
                    xs = (cx0, cx1)
                    ys = (cy0, cy1)
                    zs = (cz0 * rp1, cz1 * rp1)
                row = g // 8
                col = (g % 8) * 16
                k = 0
                for dx in (0, 1):
                    wxy2 = (w00, w01) if dx == 0 else (w10, w11)
                    for dy in (0, 1):
                        wxy = wxy2[dy]
                        for dz in (0, 1):
                            if hashed:
                                h = xs[dx] ^ ys[dy] ^ zs[dz]
                                idx = (h.astype(jnp.uint32) & HASH_MASK).astype(jnp.int32)
                            else:
                                idx = (ys[dy] + zs[dz]) * rp1 + xs[dx]
                            e0 = idx * 2 + off
                            idxb[2 * k + row, pl.ds(col, 16)] = e0
                            idxb[16 + 2 * k + row, pl.ds(col, 16)] = e0 + 1
                            w = wxy * (wz1 if dz else wz0)
                            wb[pl.ds(k * C + s, 16)] = w
                            k += 1
                return carry

            lax.fori_loop(0, G, pass_a, 0)

            descs = [
                pltpu.make_async_copy(
                    tables_hbm.at[idxb.at[j]],
                    rows.at[pl.ds(j * SLAB, SLAB)],
                    sem,
                )
                for j in range(NSLAB)
            ]
            for d in descs:
                d.start()
            for d in descs:
                d.wait()

            def pass_b(g, carry):
                s = g * 16
                acc0 = jnp.zeros((16,), jnp.float32)
                acc1 = jnp.zeros((16,), jnp.float32)
                for k in range(8):
                    f0 = rows[pl.ds(k * C + s, 16)]
                    f1 = rows[pl.ds(8 * C + k * C + s, 16)]
                    wv = wb[pl.ds(k * C + s, 16)]
                    acc0 = acc0 + wv * f0
                    acc1 = acc1 + wv * f1
                riv = iota16 + s
                plsc.store_scatter(tstage, [riv, iota16 * 0 + (2 * l)], acc0)
                plsc.store_scatter(tstage, [riv, iota16 * 0 + (2 * l + 1)], acc1)
                return carry

            lax.fori_loop(0, G, pass_b, 0)

        def proj_body(p, carry):
            elem_base = p * (2 * TOTAL_PAD)
            pltpu.sync_copy(tables_hbm.at[pl.ds(elem_base, DENSE_ELEMS)], dense)

            def chunk_body(ci, carry2):
                pbase = wid * PPW + ci * C
                pltpu.sync_copy(coords_hbm.at[pl.ds(p * 3, 3), pl.ds(pbase, C)], cb)
                for l in range(N_DENSE_LOCAL):
                    level_dense_local(l)
                for l in range(N_DENSE_LOCAL, L):
                    level_streamed(l, elem_base)
                pltpu.sync_copy(
                    tstage,
                    out_hbm.at[pl.ds(pbase, C), pl.ds(p * (2 * L), 2 * L)],
                )
                return carry2

            lax.fori_loop(0, CHUNKS, chunk_body, 0)
            return carry

        lax.fori_loop(0, 3, proj_body, 0)

    return enc(coords, tables)


def _mm_body(t_ref, w_ref, b_ref, o_ref):
    o_ref[...] = (
        jnp.dot(t_ref[...], w_ref[...], preferred_element_type=jnp.float32)
        + b_ref[...]
    )


def _decode(temporal, W, b):
    BM = 2048
    d_in = 3 * L * F
    return pl.pallas_call(
        _mm_body,
        grid=(N // BM,),
        in_specs=[
            pl.BlockSpec((BM, d_in), lambda i: (i, 0)),
            pl.BlockSpec((d_in, 64), lambda i: (0, 0)),
            pl.BlockSpec((1, 64), lambda i: (0, 0)),
        ],
        out_specs=pl.BlockSpec((BM, 64), lambda i: (i, 0)),
        out_shape=jax.ShapeDtypeStruct((N, 64), jnp.float32),
    )(temporal, W, b.reshape(1, 64))


def kernel(in_tensor, xyt_table, yzt_table, xzt_table, W, b):
    coords = jnp.concatenate(
        [in_tensor[:, (0, 1, 3)], in_tensor[:, (1, 2, 3)], in_tensor[:, (0, 2, 3)]],
        axis=1,
    ).T
    pad = jnp.zeros((TOTAL_PAD - TOTAL, F), jnp.float32)
    tables = jnp.concatenate(
        [xyt_table, pad, yzt_table, pad, xzt_table, pad], axis=0
    ).reshape(-1)
    temporal = _sc_encode(coords, tables)
    decoded = _decode(temporal, W, b)
    return (decoded, temporal)


# speedup vs baseline: 1.2608x; 1.0827x over previous
import functools

import jax
import jax.numpy as jnp
import numpy as np
from jax import lax
from jax.experimental import pallas as pl
from jax.experimental.pallas import tpu as pltpu
from jax.experimental.pallas import tpu_sc as plsc

N = 524288
L = 16
F = 2
HASH_MASK = np.uint32(2**19 - 1)
P1 = np.uint32(2654435761)
P2 = np.uint32(805459861)
RES = [8, 11, 16, 24, 35, 50, 73, 106, 153, 222, 322, 466, 675, 977, 1415, 2048]
OFF = [0, 729, 2457, 7370, 22995, 69651, 202302, 607526, 1131814, 1656102,
       2180390, 2704678, 3228966, 3753254, 4277542, 4801830]
TOTAL = 5326118
TOTAL_PAD = 5326120
N_DENSE_LOCAL = 4
DENSE_ROWS = OFF[N_DENSE_LOCAL]
DENSE_ELEMS = 2 * DENSE_ROWS + 2

NC = 2
NS = 16
NW = NC * NS
PPW = N // NW
C = 256
G = C // 16
CHUNKS = PPW // C
SLAB = 128
NSLAB = (2 * 8 * C) // SLAB


def _corner_prep(xv, yv, zv, r):
    rf = float(r)
    out = []
    for v in (xv, yv, zv):
        v01 = jnp.clip((v + 1.0) * 0.5, 0.0, 1.0)
        pv = v01 * rf
        iv = pv.astype(jnp.int32)
        fv = pv - iv.astype(jnp.float32)
        c0 = iv
        c1 = jnp.minimum(iv + 1, r)
        out.append((c0, c1, 1.0 - fv, fv))
    return out


def _sc_encode(coords, tables):
    mesh = plsc.VectorSubcoreMesh(core_axis_name="c", subcore_axis_name="s",
                                  num_cores=NC, num_subcores=NS)

    @functools.partial(
        pl.kernel,
        out_type=jax.ShapeDtypeStruct((N, 3 * L * F), jnp.float32),
        mesh=mesh,
        scratch_types=[
            pltpu.VMEM((3, C), jnp.float32),
            pltpu.VMEM((DENSE_ELEMS,), jnp.float32),
            pltpu.VMEM((NSLAB, SLAB), jnp.int32),
            pltpu.VMEM((8 * C,), jnp.float32),
            pltpu.VMEM((2 * 8 * C,), jnp.float32),
            pltpu.VMEM((C, 2 * L), jnp.float32),
            pltpu.SemaphoreType.DMA,
        ],
        compiler_params=pltpu.CompilerParams(
            use_tc_tiling_on_sc=False, needs_layout_passes=False
        ),
    )
    def enc(coords_hbm, tables_hbm, out_hbm, cb, dense, idxb, wb, rows, tstage, sem):
        wid = lax.axis_index("s") * NC + lax.axis_index("c")
        iota16 = lax.iota(jnp.int32, 16)

        def level_dense_local(l):
            r = RES[l]
            rp1 = r + 1
            off2 = 2 * OFF[l]

            def body(g, carry):
                s = g * 16
                xv = cb[0, pl.ds(s, 16)]
                yv = cb[1, pl.ds(s, 16)]
                zv = cb[2, pl.ds(s, 16)]
                (cx0, cx1, wx0, wx1), (cy0, cy1, wy0, wy1), (cz0, cz1, wz0, wz1) = \
                    _corner_prep(xv, yv, zv, r)
                az0 = cz0 * rp1
                az1 = cz1 * rp1
                w00 = wx0 * wy0
                w01 = wx0 * wy1
                w10 = wx1 * wy0
                w11 = wx1 * wy1
                acc0 = jnp.zeros((16,), jnp.float32)
                acc1 = jnp.zeros((16,), jnp.float32)
                for (cx, wxy2) in ((cx0, (w00, w01)), (cx1, (w10, w11))):
                    for (cy, wxy) in ((cy0, wxy2[0]), (cy1, wxy2[1])):
                        for (az, wz) in ((az0, wz0), (az1, wz1)):
                            e0 = ((cy + az) * rp1 + cx) * 2 + off2
                            f0 = plsc.load_gather(dense, [e0])
                            f1 = plsc.load_gather(dense, [e0 + 1])
                            w = wxy * wz
                            acc0 = acc0 + w * f0
                            acc1 = acc1 + w * f1
                riv = iota16 + s
                plsc.store_scatter(tstage, [riv, iota16 * 0 + (2 * l)], acc0)
                plsc.store_scatter(tstage, [riv, iota16 * 0 + (2 * l + 1)], acc1)
                return carry

            lax.fori_loop(0, G, body, 0)

        def level_streamed(l, elem_base):
            r = RES[l]
            rp1 = r + 1
            hashed = (rp1 ** 3) > 2**19

            def pass_a(g, carry):
                s = g * 16
                xv = cb[0, pl.ds(s, 16)]
                yv = cb[1, pl.ds(s, 16)]
                zv = cb[2, pl.ds(s, 16)]
                (cx0, cx1, wx0, wx1), (cy0, cy1, wy0, wy1), (cz0, cz1, wz0, wz1) = \
                    _corner_prep(xv, yv, zv, r)
                w00 = wx0 * wy0
                w01 = wx0 * wy1
                w10 = wx1 * wy0
                w11 = wx1 * wy1
                off = elem_base + 2 * OFF[l]
                if hashed:
                    xs = (cx0, cx1)
                    ys = ((cy0.astype(jnp.uint32) * P1).astype(jnp.int32),
                          (cy1.astype(jnp.uint32) * P1).astype(jnp.int32))
                    zs = ((cz0.astype(jnp.uint32) * P2).astype(jnp.int32),
                          (cz1.astype(jnp.uint32) * P2).astype(jnp.int32))
                else:
                    xs = (cx0, cx1)
                    ys = (cy0, cy1)
                    zs = (cz0 * rp1, cz1 * rp1)
                row = g // 8
                col = (g % 8) * 16
                k = 0
                for dx in (0, 1):
                    wxy2 = (w00, w01) if dx == 0 else (w10, w11)
                    for dy in (0, 1):
                        wxy = wxy2[dy]
                        for dz in (0, 1):
                            if hashed:
                                h = xs[dx] ^ ys[dy] ^ zs[dz]
                                idx = (h.astype(jnp.uint32) & HASH_MASK).astype(jnp.int32)
                            else:
                                idx = (ys[dy] + zs[dz]) * rp1 + xs[dx]
                            e0 = idx * 2 + off
                            idxb[2 * k + row, pl.ds(col, 16)] = e0
                            idxb[16 + 2 * k + row, pl.ds(col, 16)] = e0 + 1
                            w = wxy * (wz1 if dz else wz0)
                            wb[pl.ds(k * C + s, 16)] = w
                            k += 1
                return carry

            lax.fori_loop(0, G, pass_a, 0)

            descs = [
                pltpu.make_async_copy(
                    tables_hbm.at[idxb.at[j]],
                    rows.at[pl.ds(j * SLAB, SLAB)],
                    sem,
                )
                for j in range(NSLAB)
            ]
            for d in descs:
                d.start()
            for d in descs:
                d.wait()

            def pass_b(g, carry):
                s = g * 16
                acc0 = jnp.zeros((16,), jnp.float32)
                acc1 = jnp.zeros((16,), jnp.float32)
                for k in range(8):
                    f0 = rows[pl.ds(k * C + s, 16)]
                    f1 = rows[pl.ds(8 * C + k * C + s, 16)]
                    wv = wb[pl.ds(k * C + s, 16)]
                    acc0 = acc0 + wv * f0
                    acc1 = acc1 + wv * f1
                riv = iota16 + s
                plsc.store_scatter(tstage, [riv, iota16 * 0 + (2 * l)], acc0)
                plsc.store_scatter(tstage, [riv, iota16 * 0 + (2 * l + 1)], acc1)
                return carry

            lax.fori_loop(0, G, pass_b, 0)

        def proj_body(p, carry):
            elem_base = p * (2 * TOTAL_PAD)
            pltpu.sync_copy(tables_hbm.at[pl.ds(elem_base, DENSE_ELEMS)], dense)

            def chunk_body(ci, carry2):
                pbase = wid * PPW + ci * C
                pltpu.sync_copy(coords_hbm.at[pl.ds(p * 3, 3), pl.ds(pbase, C)], cb)
                for l in range(N_DENSE_LOCAL):
                    level_dense_local(l)
                for l in range(N_DENSE_LOCAL, L):
                    level_streamed(l, elem_base)
                pltpu.sync_copy(
                    tstage,
                    out_hbm.at[pl.ds(pbase, C), pl.ds(p * (2 * L), 2 * L)],
                )
                return carry2

            lax.fori_loop(0, CHUNKS, chunk_body, 0)
            return carry

        lax.fori_loop(0, 3, proj_body, 0)

    return enc(coords, tables)


def _mm_body(t_ref, w_ref, b_ref, o_ref):
    o_ref[...] = (
        jnp.dot(t_ref[...], w_ref[...], preferred_element_type=jnp.float32)
        + b_ref[...]
    )


def _decode(temporal, W, b):
    BM = 2048
    d_in = 3 * L * F
    return pl.pallas_call(
        _mm_body,
        grid=(N // BM,),
        in_specs=[
            pl.BlockSpec((BM, d_in), lambda i: (i, 0)),
            pl.BlockSpec((d_in, 64), lambda i: (0, 0)),
            pl.BlockSpec((1, 64), lambda i: (0, 0)),
        ],
        out_specs=pl.BlockSpec((BM, 64), lambda i: (i, 0)),
        out_shape=jax.ShapeDtypeStruct((N, 64), jnp.float32),
    )(temporal, W, b.reshape(1, 64))


def kernel(in_tensor, xyt_table, yzt_table, xzt_table, W, b):
    coords = jnp.concatenate(
        [in_tensor[:, (0, 1, 3)], in_tensor[:, (1, 2, 3)], in_tensor[:, (0, 2, 3)]],
        axis=1,
    ).T
    pad = jnp.zeros(((TOTAL_PAD - TOTAL) * F,), jnp.float32)
    tables = jnp.concatenate(
        [xyt_table.reshape(-1), pad, yzt_table.reshape(-1), pad,
         xzt_table.reshape(-1), pad]
    )
    temporal = _sc_encode(coords, tables)
    decoded = _decode(temporal, W, b)
    return (decoded, temporal)


# speedup vs baseline: 1.5044x; 1.1933x over previous
import functools

import jax
import jax.numpy as jnp
import numpy as np
from jax import lax
from jax.experimental import pallas as pl
from jax.experimental.pallas import tpu as pltpu
from jax.experimental.pallas import tpu_sc as plsc

N = 524288
L = 16
F = 2
HASH_MASK = np.uint32(2**19 - 1)
P1 = np.uint32(2654435761)
P2 = np.uint32(805459861)
RES = [8, 11, 16, 24, 35, 50, 73, 106, 153, 222, 322, 466, 675, 977, 1415, 2048]
OFF = [0, 729, 2457, 7370, 22995, 69651, 202302, 607526, 1131814, 1656102,
       2180390, 2704678, 3228966, 3753254, 4277542, 4801830]
N_DENSE_LOCAL = 4
DENSE_ROWS = OFF[N_DENSE_LOCAL]
DENSE_ELEMS = 2 * DENSE_ROWS + 2

NC = 2
NS = 16
NW = NC * NS
PPW = N // NW
C = 256
G = C // 16
CHUNKS = PPW // C
SLAB = 128
NSLAB = (2 * 8 * C) // SLAB


def _corner_prep(xv, yv, zv, r):
    rf = float(r)
    out = []
    for v in (xv, yv, zv):
        v01 = jnp.clip((v + 1.0) * 0.5, 0.0, 1.0)
        pv = v01 * rf
        iv = pv.astype(jnp.int32)
        fv = pv - iv.astype(jnp.float32)
        c0 = iv
        c1 = jnp.minimum(iv + 1, r)
        out.append((c0, c1, 1.0 - fv, fv))
    return out


def _sc_encode(coords, t0, t1, t2, d0, d1, d2):
    mesh = plsc.VectorSubcoreMesh(core_axis_name="c", subcore_axis_name="s",
                                  num_cores=NC, num_subcores=NS)

    @functools.partial(
        pl.kernel,
        out_type=jax.ShapeDtypeStruct((N, 3 * L * F), jnp.float32),
        mesh=mesh,
        scratch_types=[
            pltpu.VMEM((3, C), jnp.float32),
            pltpu.VMEM((DENSE_ELEMS,), jnp.float32),
            pltpu.VMEM((NSLAB, SLAB), jnp.int32),
            pltpu.VMEM((NSLAB, SLAB), jnp.int32),
            pltpu.VMEM((8 * C,), jnp.float32),
            pltpu.VMEM((8 * C,), jnp.float32),
            pltpu.VMEM((2 * 8 * C,), jnp.float32),
            pltpu.VMEM((2 * 8 * C,), jnp.float32),
            pltpu.VMEM((C, 2 * L), jnp.float32),
            pltpu.SemaphoreType.DMA,
            pltpu.SemaphoreType.DMA,
        ],
        compiler_params=pltpu.CompilerParams(
            use_tc_tiling_on_sc=False, needs_layout_passes=False
        ),
    )
    def enc(coords_hbm, t0h, t1h, t2h, d0h, d1h, d2h, out_hbm,
            cb, dense, idxb0, idxb1, wb0, wb1, rows0, rows1, tstage,
            sem0, sem1):
        wid = lax.axis_index("s") * NC + lax.axis_index("c")
        iota16 = lax.iota(jnp.int32, 16)
        tabs = (t0h, t1h, t2h)
        dtabs = (d0h, d1h, d2h)
        idxbs = (idxb0, idxb1)
        wbs = (wb0, wb1)
        rowss = (rows0, rows1)
        sems = (sem0, sem1)

        def level_dense_local(l):
            r = RES[l]
            rp1 = r + 1
            off2 = 2 * OFF[l]

            def body(g, carry):
                s = g * 16
                xv = cb[0, pl.ds(s, 16)]
                yv = cb[1, pl.ds(s, 16)]
                zv = cb[2, pl.ds(s, 16)]
                (cx0, cx1, wx0, wx1), (cy0, cy1, wy0, wy1), (cz0, cz1, wz0, wz1) = \
                    _corner_prep(xv, yv, zv, r)
                az0 = cz0 * rp1
                az1 = cz1 * rp1
                w00 = wx0 * wy0
                w01 = wx0 * wy1
                w10 = wx1 * wy0
                w11 = wx1 * wy1
                acc0 = jnp.zeros((16,), jnp.float32)
                acc1 = jnp.zeros((16,), jnp.float32)
                for (cx, wxy2) in ((cx0, (w00, w01)), (cx1, (w10, w11))):
                    for (cy, wxy) in ((cy0, wxy2[0]), (cy1, wxy2[1])):
                        for (az, wz) in ((az0, wz0), (az1, wz1)):
                            e0 = ((cy + az) * rp1 + cx) * 2 + off2
                            f0 = plsc.load_gather(dense, [e0])
                            f1 = plsc.load_gather(dense, [e0 + 1])
                            w = wxy * wz
                            acc0 = acc0 + w * f0
                            acc1 = acc1 + w * f1
                riv = iota16 + s
                plsc.store_scatter(tstage, [riv, iota16 * 0 + (2 * l)], acc0)
                plsc.store_scatter(tstage, [riv, iota16 * 0 + (2 * l + 1)], acc1)
                return carry

            lax.fori_loop(0, G, body, 0)

        def pass_a(l, ph):
            r = RES[l]
            rp1 = r + 1
            hashed = (rp1 ** 3) > 2**19
            idxb = idxbs[ph]
            wb = wbs[ph]

            def body(g, carry):
                s = g * 16
                xv = cb[0, pl.ds(s, 16)]
                yv = cb[1, pl.ds(s, 16)]
                zv = cb[2, pl.ds(s, 16)]
                (cx0, cx1, wx0, wx1), (cy0, cy1, wy0, wy1), (cz0, cz1, wz0, wz1) = \
                    _corner_prep(xv, yv, zv, r)
                w00 = wx0 * wy0
                w01 = wx0 * wy1
                w10 = wx1 * wy0
                w11 = wx1 * wy1
                if hashed:
                    xs = (cx0, cx1)
                    ys = ((cy0.astype(jnp.uint32) * P1).astype(jnp.int32),
                          (cy1.astype(jnp.uint32) * P1).astype(jnp.int32))
                    zs = ((cz0.astype(jnp.uint32) * P2).astype(jnp.int32),
                          (cz1.astype(jnp.uint32) * P2).astype(jnp.int32))
                else:
                    xs = (cx0, cx1)
                    ys = (cy0, cy1)
                    zs = (cz0 * rp1, cz1 * rp1)
                row = g // 8
                col = (g % 8) * 16
                k = 0
                for dx in (0, 1):
                    wxy2 = (w00, w01) if dx == 0 else (w10, w11)
                    for dy in (0, 1):
                        wxy = wxy2[dy]
                        for dz in (0, 1):
                            if hashed:
                                h = xs[dx] ^ ys[dy] ^ zs[dz]
                                idx = (h.astype(jnp.uint32) & HASH_MASK).astype(jnp.int32)
                            else:
                                idx = (ys[dy] + zs[dz]) * rp1 + xs[dx]
                            e0 = (idx + OFF[l]) * 2
                            idxb[2 * k + row, pl.ds(col, 16)] = e0
                            idxb[16 + 2 * k + row, pl.ds(col, 16)] = e0 + 1
                            w = wxy * (wz1 if dz else wz0)
                            wb[pl.ds(k * C + s, 16)] = w
                            k += 1
                return carry

            lax.fori_loop(0, G, body, 0)

        def _descs(ph, pp):
            return [
                pltpu.make_async_copy(
                    tabs[pp].at[idxbs[ph].at[j]],
                    rowss[ph].at[pl.ds(j * SLAB, SLAB)],
                    sems[ph],
                )
                for j in range(NSLAB)
            ]

        def fire(ph, p):
            for pp in range(3):
                @pl.when(p == pp)
                def _(pp=pp):
                    for d in _descs(ph, pp):
                        d.start()

        def drain(ph):
            for d in _descs(ph, 0):
                d.wait()

        def pass_b(l, ph):
            rows = rowss[ph]
            wb = wbs[ph]

            def body(g, carry):
                s = g * 16
                acc0 = jnp.zeros((16,), jnp.float32)
                acc1 = jnp.zeros((16,), jnp.float32)
                for k in range(8):
                    f0 = rows[pl.ds(k * C + s, 16)]
                    f1 = rows[pl.ds(8 * C + k * C + s, 16)]
                    wv = wb[pl.ds(k * C + s, 16)]
                    acc0 = acc0 + wv * f0
                    acc1 = acc1 + wv * f1
                riv = iota16 + s
                plsc.store_scatter(tstage, [riv, iota16 * 0 + (2 * l)], acc0)
                plsc.store_scatter(tstage, [riv, iota16 * 0 + (2 * l + 1)], acc1)
                return carry

            lax.fori_loop(0, G, body, 0)

        def proj_body(p, carry):
            for pp in range(3):
                @pl.when(p == pp)
                def _(pp=pp):
                    pltpu.sync_copy(dtabs[pp], dense)

            def chunk_body(ci, carry2):
                pbase = wid * PPW + ci * C
                pltpu.sync_copy(coords_hbm.at[pl.ds(p * 3, 3), pl.ds(pbase, C)], cb)
                pass_a(N_DENSE_LOCAL, N_DENSE_LOCAL % 2)
                fire(N_DENSE_LOCAL % 2, p)
                for l in range(N_DENSE_LOCAL):
                    level_dense_local(l)
                for l in range(N_DENSE_LOCAL + 1, L):
                    ph = l % 2
                    pass_a(l, ph)
                    fire(ph, p)
                    drain(1 - ph)
                    pass_b(l - 1, 1 - ph)
                drain((L - 1) % 2)
                pass_b(L - 1, (L - 1) % 2)
                pltpu.sync_copy(
                    tstage,
                    out_hbm.at[pl.ds(pbase, C), pl.ds(p * (2 * L), 2 * L)],
                )
                return carry2

            lax.fori_loop(0, CHUNKS, chunk_body, 0)
            return carry

        lax.fori_loop(0, 3, proj_body, 0)

    return enc(coords, t0, t1, t2, d0, d1, d2)


def _mm_body(t_ref, w_ref, b_ref, o_ref):
    o_ref[...] = (
        jnp.dot(t_ref[...], w_ref[...], preferred_element_type=jnp.float32)
        + b_ref[...]
    )


def _decode(temporal, W, b):
    BM = 2048
    d_in = 3 * L * F
    return pl.pallas_call(
        _mm_body,
        grid=(N // BM,),
        in_specs=[
            pl.BlockSpec((BM, d_in), lambda i: (i, 0)),
            pl.BlockSpec((d_in, 64), lambda i: (0, 0)),
            pl.BlockSpec((1, 64), lambda i: (0, 0)),
        ],
        out_specs=pl.BlockSpec((BM, 64), lambda i: (i, 0)),
        out_shape=jax.ShapeDtypeStruct((N, 64), jnp.float32),
    )(temporal, W, b.reshape(1, 64))


def kernel(in_tensor, xyt_table, yzt_table, xzt_table, W, b):
    coords = jnp.concatenate(
        [in_tensor[:, (0, 1, 3)], in_tensor[:, (1, 2, 3)], in_tensor[:, (0, 2, 3)]],
        axis=1,
    ).T
    f0 = xyt_table.reshape(-1)
    f1 = yzt_table.reshape(-1)
    f2 = xzt_table.reshape(-1)
    temporal = _sc_encode(coords, f0, f1, f2,
                          f0[:DENSE_ELEMS], f1[:DENSE_ELEMS], f2[:DENSE_ELEMS])
    decoded = _decode(temporal, W, b)
    return (decoded, temporal)


# speedup vs baseline: 2.3502x; 1.5622x over previous
import functools

import jax
import jax.numpy as jnp
import numpy as np
from jax import lax
from jax.experimental import pallas as pl
from jax.experimental.pallas import tpu as pltpu
from jax.experimental.pallas import tpu_sc as plsc

N = 524288
L = 16
F = 2
HASH_MASK = np.uint32(2**19 - 1)
P1 = np.uint32(2654435761)
P2 = np.uint32(805459861)
RES = [8, 11, 16, 24, 35, 50, 73, 106, 153, 222, 322, 466, 675, 977, 1415, 2048]
OFF = [0, 729, 2457, 7370, 22995, 69651, 202302, 607526, 1131814, 1656102,
       2180390, 2704678, 3228966, 3753254, 4277542, 4801830]
N_DENSE_LOCAL = 4
DENSE_ROWS = OFF[N_DENSE_LOCAL]
DENSE_ELEMS = 2 * DENSE_ROWS + 2

NC = 2
NS = 16
NW = NC * NS
PPW = N // NW
C = 256
G = C // 16
CHUNKS = PPW // C
SLAB = 128
NSLAB = (2 * 8 * C) // SLAB


TOTAL = 5326118
TOTAL_PAD = 5326120
FLAT_RC = 512
FLAT_FULL = TOTAL // FLAT_RC
FLAT_REM = TOTAL - FLAT_FULL * FLAT_RC


def _sc_flatten(t0, t1, t2):
    mesh = plsc.VectorSubcoreMesh(core_axis_name="c", subcore_axis_name="s",
                                  num_cores=NC, num_subcores=NS)
    n_iter = (FLAT_FULL + NW - 1) // NW

    @functools.partial(
        pl.kernel,
        out_type=[jax.ShapeDtypeStruct((2 * TOTAL_PAD,), jnp.float32)] * 3,
        mesh=mesh,
        scratch_types=[
            pltpu.VMEM((FLAT_RC, F), jnp.float32),
            pltpu.VMEM((2 * FLAT_RC,), jnp.float32),
        ],
        compiler_params=pltpu.CompilerParams(needs_layout_passes=False),
    )
    def flt(t0h, t1h, t2h, o0h, o1h, o2h, vbuf, fbuf):
        wid = lax.axis_index("s") * NC + lax.axis_index("c")
        iota16 = lax.iota(jnp.int32, 16)

        def repack(nrows):
            def body(j, carry):
                le = iota16 + j * 16
                v = plsc.load_gather(
                    vbuf, [lax.shift_right_logical(le, 1), le & 1])
                fbuf[pl.ds(j * 16, 16)] = v
                return carry
            lax.fori_loop(0, -((-2 * nrows) // 16), body, 0)

        for th, oh in ((t0h, o0h), (t1h, o1h), (t2h, o2h)):
            def it_body(it, carry, th=th, oh=oh):
                cid = it * NW + wid

                @pl.when(cid < FLAT_FULL)
                def _():
                    rbase = cid * FLAT_RC
                    pltpu.sync_copy(th.at[pl.ds(rbase, FLAT_RC)], vbuf)
                    repack(FLAT_RC)
                    pltpu.sync_copy(fbuf, oh.at[pl.ds(2 * rbase, 2 * FLAT_RC)])
                return carry

            lax.fori_loop(0, n_iter, it_body, 0)

            @pl.when(wid == 0)
            def _(th=th, oh=oh):
                rbase = FLAT_FULL * FLAT_RC
                pltpu.sync_copy(th.at[pl.ds(rbase, FLAT_REM)],
                                vbuf.at[pl.ds(0, FLAT_REM)])
                repack(FLAT_REM)
                pltpu.sync_copy(fbuf.at[pl.ds(0, 2 * FLAT_REM)],
                                oh.at[pl.ds(2 * rbase, 2 * FLAT_REM)])

    return flt(t0, t1, t2)


def _corner_prep(xv, yv, zv, r):
    rf = float(r)
    out = []
    for v in (xv, yv, zv):
        v01 = jnp.clip((v + 1.0) * 0.5, 0.0, 1.0)
        pv = v01 * rf
        iv = pv.astype(jnp.int32)
        fv = pv - iv.astype(jnp.float32)
        c0 = iv
        c1 = jnp.minimum(iv + 1, r)
        out.append((c0, c1, 1.0 - fv, fv))
    return out


def _sc_encode(coords, t0, t1, t2, d0, d1, d2):
    mesh = plsc.VectorSubcoreMesh(core_axis_name="c", subcore_axis_name="s",
                                  num_cores=NC, num_subcores=NS)

    @functools.partial(
        pl.kernel,
        out_type=jax.ShapeDtypeStruct((N, 3 * L * F), jnp.float32),
        mesh=mesh,
        scratch_types=[
            pltpu.VMEM((3, C), jnp.float32),
            pltpu.VMEM((DENSE_ELEMS,), jnp.float32),
            pltpu.VMEM((NSLAB, SLAB), jnp.int32),
            pltpu.VMEM((NSLAB, SLAB), jnp.int32),
            pltpu.VMEM((8 * C,), jnp.float32),
            pltpu.VMEM((8 * C,), jnp.float32),
            pltpu.VMEM((2 * 8 * C,), jnp.float32),
            pltpu.VMEM((2 * 8 * C,), jnp.float32),
            pltpu.VMEM((C, 2 * L), jnp.float32),
            pltpu.SemaphoreType.DMA,
            pltpu.SemaphoreType.DMA,
        ],
        compiler_params=pltpu.CompilerParams(
            use_tc_tiling_on_sc=False, needs_layout_passes=False
        ),
    )
    def enc(coords_hbm, t0h, t1h, t2h, d0h, d1h, d2h, out_hbm,
            cb, dense, idxb0, idxb1, wb0, wb1, rows0, rows1, tstage,
            sem0, sem1):
        wid = lax.axis_index("s") * NC + lax.axis_index("c")
        iota16 = lax.iota(jnp.int32, 16)
        tabs = (t0h, t1h, t2h)
        dtabs = (d0h, d1h, d2h)
        idxbs = (idxb0, idxb1)
        wbs = (wb0, wb1)
        rowss = (rows0, rows1)
        sems = (sem0, sem1)

        def level_dense_local(l):
            r = RES[l]
            rp1 = r + 1
            off2 = 2 * OFF[l]

            def body(g, carry):
                s = g * 16
                xv = cb[0, pl.ds(s, 16)]
                yv = cb[1, pl.ds(s, 16)]
                zv = cb[2, pl.ds(s, 16)]
                (cx0, cx1, wx0, wx1), (cy0, cy1, wy0, wy1), (cz0, cz1, wz0, wz1) = \
                    _corner_prep(xv, yv, zv, r)
                az0 = cz0 * rp1
                az1 = cz1 * rp1
                w00 = wx0 * wy0
                w01 = wx0 * wy1
                w10 = wx1 * wy0
                w11 = wx1 * wy1
                acc0 = jnp.zeros((16,), jnp.float32)
                acc1 = jnp.zeros((16,), jnp.float32)
                for (cx, wxy2) in ((cx0, (w00, w01)), (cx1, (w10, w11))):
                    for (cy, wxy) in ((cy0, wxy2[0]), (cy1, wxy2[1])):
                        for (az, wz) in ((az0, wz0), (az1, wz1)):
                            e0 = ((cy + az) * rp1 + cx) * 2 + off2
                            f0 = plsc.load_gather(dense, [e0])
                            f1 = plsc.load_gather(dense, [e0 + 1])
                            w = wxy * wz
                            acc0 = acc0 + w * f0
                            acc1 = acc1 + w * f1
                riv = iota16 + s
                plsc.store_scatter(tstage, [riv, iota16 * 0 + (2 * l)], acc0)
                plsc.store_scatter(tstage, [riv, iota16 * 0 + (2 * l + 1)], acc1)
                return carry

            lax.fori_loop(0, G, body, 0)

        def pass_a(l, ph):
            r = RES[l]
            rp1 = r + 1
            hashed = (rp1 ** 3) > 2**19
            idxb = idxbs[ph]
            wb = wbs[ph]

            def body(g, carry):
                s = g * 16
                xv = cb[0, pl.ds(s, 16)]
                yv = cb[1, pl.ds(s, 16)]
                zv = cb[2, pl.ds(s, 16)]
                (cx0, cx1, wx0, wx1), (cy0, cy1, wy0, wy1), (cz0, cz1, wz0, wz1) = \
                    _corner_prep(xv, yv, zv, r)
                w00 = wx0 * wy0
                w01 = wx0 * wy1
                w10 = wx1 * wy0
                w11 = wx1 * wy1
                if hashed:
                    xs = (cx0, cx1)
                    ys = ((cy0.astype(jnp.uint32) * P1).astype(jnp.int32),
                          (cy1.astype(jnp.uint32) * P1).astype(jnp.int32))
                    zs = ((cz0.astype(jnp.uint32) * P2).astype(jnp.int32),
                          (cz1.astype(jnp.uint32) * P2).astype(jnp.int32))
                else:
                    xs = (cx0, cx1)
                    ys = (cy0, cy1)
                    zs = (cz0 * rp1, cz1 * rp1)
                row = g // 8
                col = (g % 8) * 16
                k = 0
                for dx in (0, 1):
                    wxy2 = (w00, w01) if dx == 0 else (w10, w11)
                    for dy in (0, 1):
                        wxy = wxy2[dy]
                        for dz in (0, 1):
                            if hashed:
                                h = xs[dx] ^ ys[dy] ^ zs[dz]
                                idx = (h.astype(jnp.uint32) & HASH_MASK).astype(jnp.int32)
                            else:
                                idx = (ys[dy] + zs[dz]) * rp1 + xs[dx]
                            e0 = (idx + OFF[l]) * 2
                            idxb[2 * k + row, pl.ds(col, 16)] = e0
                            idxb[16 + 2 * k + row, pl.ds(col, 16)] = e0 + 1
                            w = wxy * (wz1 if dz else wz0)
                            wb[pl.ds(k * C + s, 16)] = w
                            k += 1
                return carry

            lax.fori_loop(0, G, body, 0)

        def _descs(ph, pp):
            return [
                pltpu.make_async_copy(
                    tabs[pp].at[idxbs[ph].at[j]],
                    rowss[ph].at[pl.ds(j * SLAB, SLAB)],
                    sems[ph],
                )
                for j in range(NSLAB)
            ]

        def fire(ph, p):
            for pp in range(3):
                @pl.when(p == pp)
                def _(pp=pp):
                    for d in _descs(ph, pp):
                        d.start()

        def drain(ph):
            for d in _descs(ph, 0):
                d.wait()

        def pass_b(l, ph):
            rows = rowss[ph]
            wb = wbs[ph]

            def body(g, carry):
                s = g * 16
                acc0 = jnp.zeros((16,), jnp.float32)
                acc1 = jnp.zeros((16,), jnp.float32)
                for k in range(8):
                    f0 = rows[pl.ds(k * C + s, 16)]
                    f1 = rows[pl.ds(8 * C + k * C + s, 16)]
                    wv = wb[pl.ds(k * C + s, 16)]
                    acc0 = acc0 + wv * f0
                    acc1 = acc1 + wv * f1
                riv = iota16 + s
                plsc.store_scatter(tstage, [riv, iota16 * 0 + (2 * l)], acc0)
                plsc.store_scatter(tstage, [riv, iota16 * 0 + (2 * l + 1)], acc1)
                return carry

            lax.fori_loop(0, G, body, 0)

        def proj_body(p, carry):
            for pp in range(3):
                @pl.when(p == pp)
                def _(pp=pp):
                    pltpu.sync_copy(dtabs[pp], dense)

            def chunk_body(ci, carry2):
                pbase = wid * PPW + ci * C
                pltpu.sync_copy(coords_hbm.at[pl.ds(p * 3, 3), pl.ds(pbase, C)], cb)
                pass_a(N_DENSE_LOCAL, N_DENSE_LOCAL % 2)
                fire(N_DENSE_LOCAL % 2, p)
                for l in range(N_DENSE_LOCAL):
                    level_dense_local(l)
                for l in range(N_DENSE_LOCAL + 1, L):
                    ph = l % 2
                    pass_a(l, ph)
                    fire(ph, p)
                    drain(1 - ph)
                    pass_b(l - 1, 1 - ph)
                drain((L - 1) % 2)
                pass_b(L - 1, (L - 1) % 2)
                pltpu.sync_copy(
                    tstage,
                    out_hbm.at[pl.ds(pbase, C), pl.ds(p * (2 * L), 2 * L)],
                )
                return carry2

            lax.fori_loop(0, CHUNKS, chunk_body, 0)
            return carry

        lax.fori_loop(0, 3, proj_body, 0)

    return enc(coords, t0, t1, t2, d0, d1, d2)


def _mm_body(t_ref, w_ref, b_ref, o_ref):
    o_ref[...] = (
        jnp.dot(t_ref[...], w_ref[...], preferred_element_type=jnp.float32)
        + b_ref[...]
    )


def _decode(temporal, W, b):
    BM = 2048
    d_in = 3 * L * F
    return pl.pallas_call(
        _mm_body,
        grid=(N // BM,),
        in_specs=[
            pl.BlockSpec((BM, d_in), lambda i: (i, 0)),
            pl.BlockSpec((d_in, 64), lambda i: (0, 0)),
            pl.BlockSpec((1, 64), lambda i: (0, 0)),
        ],
        out_specs=pl.BlockSpec((BM, 64), lambda i: (i, 0)),
        out_shape=jax.ShapeDtypeStruct((N, 64), jnp.float32),
    )(temporal, W, b.reshape(1, 64))


def kernel(in_tensor, xyt_table, yzt_table, xzt_table, W, b):
    coords = jnp.concatenate(
        [in_tensor[:, (0, 1, 3)], in_tensor[:, (1, 2, 3)], in_tensor[:, (0, 2, 3)]],
        axis=1,
    ).T
    f0, f1, f2 = _sc_flatten(xyt_table, yzt_table, xzt_table)
    temporal = _sc_encode(coords, f0, f1, f2,
                          f0[:DENSE_ELEMS], f1[:DENSE_ELEMS], f2[:DENSE_ELEMS])
    decoded = _decode(temporal, W, b)
    return (decoded, temporal)


# speedup vs baseline: 2.3509x; 1.0003x over previous
import functools

import jax
import jax.numpy as jnp
import numpy as np
from jax import lax
from jax.experimental import pallas as pl
from jax.experimental.pallas import tpu as pltpu
from jax.experimental.pallas import tpu_sc as plsc

N = 524288
L = 16
F = 2
HASH_MASK = np.uint32(2**19 - 1)
P1 = np.uint32(2654435761)
P2 = np.uint32(805459861)
RES = [8, 11, 16, 24, 35, 50, 73, 106, 153, 222, 322, 466, 675, 977, 1415, 2048]
OFF = [0, 729, 2457, 7370, 22995, 69651, 202302, 607526, 1131814, 1656102,
       2180390, 2704678, 3228966, 3753254, 4277542, 4801830]
N_DENSE_LOCAL = 4
DENSE_ROWS = OFF[N_DENSE_LOCAL]
DENSE_ELEMS = 2 * DENSE_ROWS + 2

NC = 2
NS = 16
NW = NC * NS
PPW = N // NW
C = 256
G = C // 16
CHUNKS = PPW // C
SLAB = 128
NSLAB = (2 * 8 * C) // SLAB


TOTAL = 5326118
TOTAL_PAD = 5326120
FLAT_RC = 512
FLAT_FULL = TOTAL // FLAT_RC
FLAT_REM = TOTAL - FLAT_FULL * FLAT_RC


def _sc_flatten(t0, t1, t2):
    mesh = plsc.VectorSubcoreMesh(core_axis_name="c", subcore_axis_name="s",
                                  num_cores=NC, num_subcores=NS)
    n_iter = (FLAT_FULL + NW - 1) // NW

    @functools.partial(
        pl.kernel,
        out_type=[jax.ShapeDtypeStruct((2 * TOTAL_PAD,), jnp.float32)] * 3,
        mesh=mesh,
        scratch_types=[
            pltpu.VMEM((FLAT_RC, F), jnp.float32),
            pltpu.VMEM((2 * FLAT_RC,), jnp.float32),
        ],
        compiler_params=pltpu.CompilerParams(needs_layout_passes=False),
    )
    def flt(t0h, t1h, t2h, o0h, o1h, o2h, vbuf, fbuf):
        wid = lax.axis_index("s") * NC + lax.axis_index("c")
        iota16 = lax.iota(jnp.int32, 16)

        def repack(nrows):
            def body(j, carry):
                le = iota16 + j * 16
                v = plsc.load_gather(
                    vbuf, [lax.shift_right_logical(le, 1), le & 1])
                fbuf[pl.ds(j * 16, 16)] = v
                return carry
            lax.fori_loop(0, -((-2 * nrows) // 16), body, 0)

        for th, oh in ((t0h, o0h), (t1h, o1h), (t2h, o2h)):
            def it_body(it, carry, th=th, oh=oh):
                cid = it * NW + wid

                @pl.when(cid < FLAT_FULL)
                def _():
                    rbase = cid * FLAT_RC
                    pltpu.sync_copy(th.at[pl.ds(rbase, FLAT_RC)], vbuf)
                    repack(FLAT_RC)
                    pltpu.sync_copy(fbuf, oh.at[pl.ds(2 * rbase, 2 * FLAT_RC)])
                return carry

            lax.fori_loop(0, n_iter, it_body, 0)

            @pl.when(wid == 0)
            def _(th=th, oh=oh):
                rbase = FLAT_FULL * FLAT_RC
                pltpu.sync_copy(th.at[pl.ds(rbase, FLAT_REM)],
                                vbuf.at[pl.ds(0, FLAT_REM)])
                repack(FLAT_REM)
                pltpu.sync_copy(fbuf.at[pl.ds(0, 2 * FLAT_REM)],
                                oh.at[pl.ds(2 * rbase, 2 * FLAT_REM)])

    return flt(t0, t1, t2)


def _corner_prep(xv, yv, zv, r):
    rf = float(r)
    out = []
    for v in (xv, yv, zv):
        v01 = jnp.clip((v + 1.0) * 0.5, 0.0, 1.0)
        pv = v01 * rf
        iv = pv.astype(jnp.int32)
        fv = pv - iv.astype(jnp.float32)
        c0 = iv
        c1 = jnp.minimum(iv + 1, r)
        out.append((c0, c1, 1.0 - fv, fv))
    return out


def _sc_encode(coords, t0, t1, t2):
    mesh = plsc.VectorSubcoreMesh(core_axis_name="c", subcore_axis_name="s",
                                  num_cores=NC, num_subcores=NS)

    @functools.partial(
        pl.kernel,
        out_type=jax.ShapeDtypeStruct((N, 3 * L * F), jnp.float32),
        mesh=mesh,
        scratch_types=[
            pltpu.VMEM((3, C), jnp.float32),
            pltpu.VMEM((DENSE_ELEMS,), jnp.float32),
            pltpu.VMEM((NSLAB, SLAB), jnp.int32),
            pltpu.VMEM((NSLAB, SLAB), jnp.int32),
            pltpu.VMEM((8 * C,), jnp.float32),
            pltpu.VMEM((8 * C,), jnp.float32),
            pltpu.VMEM((2 * 8 * C,), jnp.float32),
            pltpu.VMEM((2 * 8 * C,), jnp.float32),
            pltpu.VMEM((C, 2 * L), jnp.float32),
            pltpu.SemaphoreType.DMA,
            pltpu.SemaphoreType.DMA,
        ],
        compiler_params=pltpu.CompilerParams(
            use_tc_tiling_on_sc=False, needs_layout_passes=False
        ),
    )
    def enc(coords_hbm, t0h, t1h, t2h, out_hbm,
            cb, dense, idxb0, idxb1, wb0, wb1, rows0, rows1, tstage,
            sem0, sem1):
        wid = lax.axis_index("s") * NC + lax.axis_index("c")
        iota16 = lax.iota(jnp.int32, 16)
        tabs = (t0h, t1h, t2h)
        idxbs = (idxb0, idxb1)
        wbs = (wb0, wb1)
        rowss = (rows0, rows1)
        sems = (sem0, sem1)

        def level_dense_local(l):
            r = RES[l]
            rp1 = r + 1
            off2 = 2 * OFF[l]

            def body(g, carry):
                s = g * 16
                xv = cb[0, pl.ds(s, 16)]
                yv = cb[1, pl.ds(s, 16)]
                zv = cb[2, pl.ds(s, 16)]
                (cx0, cx1, wx0, wx1), (cy0, cy1, wy0, wy1), (cz0, cz1, wz0, wz1) = \
                    _corner_prep(xv, yv, zv, r)
                az0 = cz0 * rp1
                az1 = cz1 * rp1
                w00 = wx0 * wy0
                w01 = wx0 * wy1
                w10 = wx1 * wy0
                w11 = wx1 * wy1
                acc0 = jnp.zeros((16,), jnp.float32)
                acc1 = jnp.zeros((16,), jnp.float32)
                for (cx, wxy2) in ((cx0, (w00, w01)), (cx1, (w10, w11))):
                    for (cy, wxy) in ((cy0, wxy2[0]), (cy1, wxy2[1])):
                        for (az, wz) in ((az0, wz0), (az1, wz1)):
                            e0 = ((cy + az) * rp1 + cx) * 2 + off2
                            f0 = plsc.load_gather(dense, [e0])
                            f1 = plsc.load_gather(dense, [e0 + 1])
                            w = wxy * wz
                            acc0 = acc0 + w * f0
                            acc1 = acc1 + w * f1
                riv = iota16 + s
                plsc.store_scatter(tstage, [riv, iota16 * 0 + (2 * l)], acc0)
                plsc.store_scatter(tstage, [riv, iota16 * 0 + (2 * l + 1)], acc1)
                return carry

            lax.fori_loop(0, G, body, 0)

        def pass_a(l, ph):
            r = RES[l]
            rp1 = r + 1
            hashed = (rp1 ** 3) > 2**19
            idxb = idxbs[ph]
            wb = wbs[ph]

            def body(g, carry):
                s = g * 16
                xv = cb[0, pl.ds(s, 16)]
                yv = cb[1, pl.ds(s, 16)]
                zv = cb[2, pl.ds(s, 16)]
                (cx0, cx1, wx0, wx1), (cy0, cy1, wy0, wy1), (cz0, cz1, wz0, wz1) = \
                    _corner_prep(xv, yv, zv, r)
                w00 = wx0 * wy0
                w01 = wx0 * wy1
                w10 = wx1 * wy0
                w11 = wx1 * wy1
                if hashed:
                    xs = (cx0, cx1)
                    ys = ((cy0.astype(jnp.uint32) * P1).astype(jnp.int32),
                          (cy1.astype(jnp.uint32) * P1).astype(jnp.int32))
                    zs = ((cz0.astype(jnp.uint32) * P2).astype(jnp.int32),
                          (cz1.astype(jnp.uint32) * P2).astype(jnp.int32))
                else:
                    xs = (cx0, cx1)
                    ys = (cy0, cy1)
                    zs = (cz0 * rp1, cz1 * rp1)
                row = g // 8
                col = (g % 8) * 16
                k = 0
                for dx in (0, 1):
                    wxy2 = (w00, w01) if dx == 0 else (w10, w11)
                    for dy in (0, 1):
                        wxy = wxy2[dy]
                        for dz in (0, 1):
                            if hashed:
                                h = xs[dx] ^ ys[dy] ^ zs[dz]
                                idx = (h.astype(jnp.uint32) & HASH_MASK).astype(jnp.int32)
                            else:
                                idx = (ys[dy] + zs[dz]) * rp1 + xs[dx]
                            e0 = (idx + OFF[l]) * 2
                            idxb[2 * k + row, pl.ds(col, 16)] = e0
                            idxb[16 + 2 * k + row, pl.ds(col, 16)] = e0 + 1
                            w = wxy * (wz1 if dz else wz0)
                            wb[pl.ds(k * C + s, 16)] = w
                            k += 1
                return carry

            lax.fori_loop(0, G, body, 0)

        def _descs(ph, pp):
            return [
                pltpu.make_async_copy(
                    tabs[pp].at[idxbs[ph].at[j]],
                    rowss[ph].at[pl.ds(j * SLAB, SLAB)],
                    sems[ph],
                )
                for j in range(NSLAB)
            ]

        def fire(ph, p):
            for pp in range(3):
                @pl.when(p == pp)
                def _(pp=pp):
                    for d in _descs(ph, pp):
                        d.start()

        def drain(ph):
            for d in _descs(ph, 0):
                d.wait()

        def pass_b(l, ph):
            rows = rowss[ph]
            wb = wbs[ph]

            def body(g, carry):
                s = g * 16
                acc0 = jnp.zeros((16,), jnp.float32)
                acc1 = jnp.zeros((16,), jnp.float32)
                for k in range(8):
                    f0 = rows[pl.ds(k * C + s, 16)]
                    f1 = rows[pl.ds(8 * C + k * C + s, 16)]
                    wv = wb[pl.ds(k * C + s, 16)]
                    acc0 = acc0 + wv * f0
                    acc1 = acc1 + wv * f1
                riv = iota16 + s
                plsc.store_scatter(tstage, [riv, iota16 * 0 + (2 * l)], acc0)
                plsc.store_scatter(tstage, [riv, iota16 * 0 + (2 * l + 1)], acc1)
                return carry

            lax.fori_loop(0, G, body, 0)

        def proj_body(p, carry):
            for pp in range(3):
                @pl.when(p == pp)
                def _(pp=pp):
                    pltpu.sync_copy(tabs[pp].at[pl.ds(0, DENSE_ELEMS)], dense)

            def chunk_body(ci, carry2):
                pbase = wid * PPW + ci * C
                pltpu.sync_copy(coords_hbm.at[pl.ds(p * 3, 3), pl.ds(pbase, C)], cb)
                pass_a(N_DENSE_LOCAL, N_DENSE_LOCAL % 2)
                fire(N_DENSE_LOCAL % 2, p)
                for l in range(N_DENSE_LOCAL):
                    level_dense_local(l)
                for l in range(N_DENSE_LOCAL + 1, L):
                    ph = l % 2
                    pass_a(l, ph)
                    fire(ph, p)
                    drain(1 - ph)
                    pass_b(l - 1, 1 - ph)
                drain((L - 1) % 2)
                pass_b(L - 1, (L - 1) % 2)
                pltpu.sync_copy(
                    tstage,
                    out_hbm.at[pl.ds(pbase, C), pl.ds(p * (2 * L), 2 * L)],
                )
                return carry2

            lax.fori_loop(0, CHUNKS, chunk_body, 0)
            return carry

        lax.fori_loop(0, 3, proj_body, 0)

    return enc(coords, t0, t1, t2)


def _mm_body(t_ref, w_ref, b_ref, o_ref):
    o_ref[...] = (
        jnp.dot(t_ref[...], w_ref[...], preferred_element_type=jnp.float32)
        + b_ref[...]
    )


def _decode(temporal, W, b):
    BM = 2048
    d_in = 3 * L * F
    return pl.pallas_call(
        _mm_body,
        grid=(N // BM,),
        in_specs=[
            pl.BlockSpec((BM, d_in), lambda i: (i, 0)),
            pl.BlockSpec((d_in, 64), lambda i: (0, 0)),
            pl.BlockSpec((1, 64), lambda i: (0, 0)),
        ],
        out_specs=pl.BlockSpec((BM, 64), lambda i: (i, 0)),
        out_shape=jax.ShapeDtypeStruct((N, 64), jnp.float32),
    )(temporal, W, b.reshape(1, 64))


def kernel(in_tensor, xyt_table, yzt_table, xzt_table, W, b):
    coords = jnp.concatenate(
        [in_tensor[:, (0, 1, 3)], in_tensor[:, (1, 2, 3)], in_tensor[:, (0, 2, 3)]],
        axis=1,
    ).T
    f0, f1, f2 = _sc_flatten(xyt_table, yzt_table, xzt_table)
    temporal = _sc_encode(coords, f0, f1, f2)
    decoded = _decode(temporal, W, b)
    return (decoded, temporal)
